# baseline (speedup 1.0000x reference)
import functools

import jax
import jax.numpy as jnp
from jax import lax
from jax.experimental import pallas as pl
from jax.experimental.pallas import tpu as pltpu
from jax.experimental.pallas import tpu_sc as plsc

_N, _D, _H, _L, _NG, _E = 10000, 128, 128, 64, 64, 320000
_HW = _H // 2
_NP = 10240
_NC, _NS = 2, 16
_NTILE = _NC * _NS
_EPT = _E // _NTILE
_K = 50
_NCHUNK = _EPT // _K
_W = 25
_NW = _NCHUNK // _W
_ROWS_PT = _NP // _NS


def _mp_round(hp, src_r, dst_r, zeros_blk):
    mesh = plsc.VectorSubcoreMesh(
        core_axis_name="c", subcore_axis_name="s",
        num_cores=_NC, num_subcores=_NS)

    @functools.partial(
        pl.kernel,
        mesh=mesh,
        out_type=jax.ShapeDtypeStruct((_NC, _NP, _H), jnp.float32),
        compiler_params=pltpu.CompilerParams(needs_layout_passes=False, use_tc_tiling_on_sc=False),
        scratch_types=[
            pltpu.VMEM((_W, _K), jnp.int32),
            pltpu.VMEM((_W, _K), jnp.int32),
            pltpu.VMEM((_W, _K), jnp.int32),
            pltpu.VMEM((_W, _K), jnp.int32),
            pltpu.VMEM((_K, _HW), jnp.int32),
            pltpu.VMEM((_K, _HW), jnp.int32),
            pltpu.VMEM((_K, _H), jnp.float32),
            pltpu.VMEM((_K, _H), jnp.float32),
            pltpu.VMEM_SHARED((_NP, _H), jnp.float32),
            pltpu.SemaphoreType.DMA,
            pltpu.SemaphoreType.DMA,
            pltpu.SemaphoreType.DMA,
            pltpu.SemaphoreType.DMA,
            pltpu.SemaphoreType.DMA,
            pltpu.SemaphoreType.DMA,
        ],
    )
    def body(h_hbm, src_hbm, dst_hbm, z_hbm, out_hbm,
             si0, si1, di0, di1, p0, p1, u0, u1, agg_sh,
             g0, g1, s0, s1, i0, i1):
        c = lax.axis_index("c")
        s = lax.axis_index("s")
        wid = s * _NC + c
        base = wid * _NW
        pltpu.async_copy(src_hbm.at[base], si0, i0)
        pltpu.async_copy(dst_hbm.at[base], di0, i0)
        pltpu.sync_copy(z_hbm, agg_sh.at[pl.ds(s * _ROWS_PT, _ROWS_PT)])
        plsc.subcore_barrier()

        def unpack(pbuf, ubuf):
            def row(r, carry):
                for g in range(_HW // 16):
                    w = pbuf[r, pl.ds(g * 16, 16)]
                    lo = plsc.bitcast(w << 16, jnp.float32)
                    hi = plsc.bitcast(w & jnp.int32(-65536), jnp.float32)
                    ubuf[r, pl.ds(g * 16, 16)] = lo
                    ubuf[r, pl.ds(_HW + g * 16, 16)] = hi
                return carry

            lax.fori_loop(0, _K, row, 0, unroll=10)

        def run_window(siw, diw, isw, nbase, nsi, ndi, nisw):
            pltpu.make_async_copy(src_hbm.at[base], siw, isw).wait()
            pltpu.make_async_copy(dst_hbm.at[base], diw, isw).wait()
            pltpu.async_copy(src_hbm.at[nbase], nsi, nisw)
            pltpu.async_copy(dst_hbm.at[nbase], ndi, nisw)

            def gath(j, pbuf, gsem):
                pltpu.async_copy(h_hbm.at[siw.at[j]], pbuf, gsem)

            def gwait(pbuf, gsem):
                pltpu.make_async_copy(h_hbm.at[siw.at[0]], pbuf, gsem).wait()

            def sscat(j, ubuf, ssem):
                pltpu.async_copy(ubuf, agg_sh.at[diw.at[j]], ssem, add=True)

            def swait(ubuf, ssem):
                pltpu.make_async_copy(
                    ubuf, agg_sh.at[diw.at[0]], ssem).wait()

            gath(0, p0, g0)
            gath(1, p1, g1)
            gwait(p0, g0)
            unpack(p0, u0)
            gath(2, p0, g0)
            sscat(0, u0, s0)
            gwait(p1, g1)
            unpack(p1, u1)
            gath(3, p1, g1)
            sscat(1, u1, s1)

            def step(i, carry):
                j = 2 * i
                gwait(p0, g0)
                swait(u0, s0)
                unpack(p0, u0)
                gath(j + 2, p0, g0)
                sscat(j, u0, s0)
                gwait(p1, g1)
                swait(u1, s1)
                unpack(p1, u1)
                gath(j + 3, p1, g1)
                sscat(j + 1, u1, s1)
                return carry

            lax.fori_loop(1, (_W - 3) // 2, step, 0)
            gwait(p0, g0)
            swait(u0, s0)
            unpack(p0, u0)
            gath(_W - 1, p0, g0)
            sscat(_W - 3, u0, s0)
            gwait(p1, g1)
            swait(u1, s1)
            unpack(p1, u1)
            sscat(_W - 2, u1, s1)
            gwait(p0, g0)
            swait(u0, s0)
            unpack(p0, u0)
            sscat(_W - 1, u0, s0)
            swait(u1, s1)
            swait(u0, s0)

        def wpair(iw, carry):
            wa = 2 * iw
            run_window(si0, di0, i0, base + wa + 1, si1, di1, i1)
            nb = base + lax.rem(wa + 2, _NW)
            run_window(si1, di1, i1, nb, si0, di0, i0)
            return carry

        lax.fori_loop(0, _NW // 2, wpair, 0)
        pltpu.make_async_copy(src_hbm.at[base], si0, i0).wait()
        pltpu.make_async_copy(dst_hbm.at[base], di0, i0).wait()
        plsc.subcore_barrier()
        pltpu.sync_copy(agg_sh.at[pl.ds(s * _ROWS_PT, _ROWS_PT)],
                        out_hbm.at[c, pl.ds(s * _ROWS_PT, _ROWS_PT)])

    return body(hp, src_r, dst_r, zeros_blk)


_BLK = 2048


def _pack_rows(acc):
    a16 = lax.bitcast_convert_type(
        acc[:, :_HW].astype(jnp.bfloat16), jnp.uint16)
    b16 = lax.bitcast_convert_type(
        acc[:, _HW:].astype(jnp.bfloat16), jnp.uint16)
    w = a16.astype(jnp.uint32) | (b16.astype(jnp.uint32) << 16)
    return lax.bitcast_convert_type(w, jnp.int32)


def _mm_relu(x, W, b):
    def body(x_ref, w_ref, b_ref, o_ref):
        acc = lax.dot_general(x_ref[...], w_ref[...],
                              (((1,), (1,)), ((), ())),
                              preferred_element_type=jnp.float32)
        o_ref[...] = _pack_rows(jnp.maximum(acc + b_ref[...], 0.0))

    return pl.pallas_call(
        body,
        grid=(_NP // _BLK,),
        in_specs=[
            pl.BlockSpec((_BLK, _D), lambda i: (i, 0)),
            pl.BlockSpec((_H, _D), lambda i: (0, 0)),
            pl.BlockSpec((1, _H), lambda i: (0, 0)),
        ],
        out_specs=pl.BlockSpec((_BLK, _HW), lambda i: (i, 0)),
        out_shape=jax.ShapeDtypeStruct((_NP, _HW), jnp.int32),
    )(x, W, b.reshape(1, _H))


def _mm_relu_sum(p, W, b, packed=True):
    def body(p_ref, w_ref, b_ref, o_ref):
        a = p_ref[0] + p_ref[1]
        acc = lax.dot_general(a, w_ref[...],
                              (((1,), (1,)), ((), ())),
                              preferred_element_type=jnp.float32)
        r = jnp.maximum(acc + b_ref[...], 0.0)
        o_ref[...] = _pack_rows(r) if packed else r

    od = _HW if packed else _H
    return pl.pallas_call(
        body,
        grid=(_NP // _BLK,),
        in_specs=[
            pl.BlockSpec((2, _BLK, _H), lambda i: (0, i, 0)),
            pl.BlockSpec((_H, _H), lambda i: (0, 0)),
            pl.BlockSpec((1, _H), lambda i: (0, 0)),
        ],
        out_specs=pl.BlockSpec((_BLK, od), lambda i: (i, 0)),
        out_shape=jax.ShapeDtypeStruct(
            (_NP, od), jnp.int32 if packed else jnp.float32),
    )(p, W, b.reshape(1, _H))


def _pool_heads(h, batch_row, Wm, bm, Wl, bl):
    def body(h_ref, b_ref, wm_ref, bm_ref, wl_ref, bl_ref, om_ref, ol_ref):
        gids = lax.broadcasted_iota(jnp.int32, (_NG, _NP), 0)
        mask = jnp.where(b_ref[...] == gids, 1.0, 0.0)
        pooled = lax.dot_general(mask, h_ref[...],
                                 (((1,), (0,)), ((), ())),
                                 preferred_element_type=jnp.float32)
        om_ref[...] = lax.dot_general(pooled, wm_ref[...],
                                      (((1,), (1,)), ((), ())),
                                      preferred_element_type=jnp.float32) + bm_ref[...]
        ol_ref[...] = lax.dot_general(pooled, wl_ref[...],
                                      (((1,), (1,)), ((), ())),
                                      preferred_element_type=jnp.float32) + bl_ref[...]

    return pl.pallas_call(
        body,
        out_shape=[jax.ShapeDtypeStruct((_NG, _L), jnp.float32),
                   jax.ShapeDtypeStruct((_NG, _L), jnp.float32)],
    )(h, batch_row, Wm, bm.reshape(1, _L), Wl, bl.reshape(1, _L))


def kernel(x, edge_index, batch, W_in, b_in, W1, b1, W2, b2, W3, b3,
           W_mean, b_mean, W_logvar, b_logvar):
    src_r = edge_index[0].reshape(_NTILE * _NW, _W, _K)
    dst_r = edge_index[1].reshape(_NTILE * _NW, _W, _K)
    zeros_blk = jnp.zeros((_ROWS_PT, _H), dtype=jnp.float32)
    x_pad = jnp.pad(x, ((0, _NP - _N), (0, 0)))
    batch_row = jnp.pad(batch.astype(jnp.int32), (0, _NP - _N),
                        constant_values=_NG).reshape(1, _NP)

    hp = _mm_relu(x_pad, W_in, b_in)
    for li, (W, b) in enumerate(((W1, b1), (W2, b2), (W3, b3))):
        p = _mp_round(hp, src_r, dst_r, zeros_blk)
        hp = _mm_relu_sum(p, W, b, packed=(li < 2))
    z_mean, z_logvar = _pool_heads(hp, batch_row, W_mean, b_mean,
                                   W_logvar, b_logvar)
    return (z_mean, z_logvar)


# speedup vs baseline: 1.8167x; 1.8167x over previous
import functools

import jax
import jax.numpy as jnp
from jax import lax
from jax.experimental import pallas as pl
from jax.experimental.pallas import tpu as pltpu
from jax.experimental.pallas import tpu_sc as plsc

_N, _D, _H, _L, _NG, _E = 10000, 128, 128, 64, 64, 320000
_NP = 10240
_NC, _NS = 2, 16
_NTILE = _NC * _NS
_EPT = _E // _NTILE
_K = 100
_NCHUNK = _EPT // _K
_W = 25
_NW = _NCHUNK // _W
_ROWS_PT = _NP // _NS


def _mp_round(h, src_r, dst_r, zeros_blk):
    mesh = plsc.VectorSubcoreMesh(
        core_axis_name="c", subcore_axis_name="s",
        num_cores=_NC, num_subcores=_NS)

    @functools.partial(
        pl.kernel,
        mesh=mesh,
        out_type=jax.ShapeDtypeStruct((_NC, _NP, _H), jnp.float32),
        scratch_types=[
            pltpu.VMEM((_W, _K), jnp.int32),
            pltpu.VMEM((_W, _K), jnp.int32),
            pltpu.VMEM((_W, _K), jnp.int32),
            pltpu.VMEM((_W, _K), jnp.int32),
            pltpu.VMEM((_K, _H), jnp.float32),
            pltpu.VMEM((_K, _H), jnp.float32),
            pltpu.VMEM_SHARED((_NP, _H), jnp.float32),
            pltpu.SemaphoreType.DMA,
            pltpu.SemaphoreType.DMA,
            pltpu.SemaphoreType.DMA,
            pltpu.SemaphoreType.DMA,
        ],
    )
    def body(h_hbm, src_hbm, dst_hbm, z_hbm, out_hbm,
             si0, si1, di0, di1, buf0, buf1, agg_sh,
             gsem0, gsem1, isem0, isem1):
        c = lax.axis_index("c")
        s = lax.axis_index("s")
        wid = s * _NC + c
        base = wid * _NW
        iwins = [(si0, di0, isem0), (si1, di1, isem1)]
        pltpu.async_copy(src_hbm.at[base], si0, isem0)
        pltpu.async_copy(dst_hbm.at[base], di0, isem0)
        pltpu.sync_copy(z_hbm, agg_sh.at[pl.ds(s * _ROWS_PT, _ROWS_PT)])
        plsc.subcore_barrier()

        for w in range(_NW):
            siw, diw, isw = iwins[w % 2]
            pltpu.make_async_copy(src_hbm.at[base], siw, isw).wait()
            pltpu.make_async_copy(dst_hbm.at[base], diw, isw).wait()
            if w + 1 < _NW:
                sin_, din_, isn = iwins[(w + 1) % 2]
                pltpu.async_copy(src_hbm.at[base + w + 1], sin_, isn)
                pltpu.async_copy(dst_hbm.at[base + w + 1], din_, isn)

            def gath(j, buf, sem):
                pltpu.async_copy(h_hbm.at[siw.at[j]], buf, sem)

            def gwait(buf, sem):
                pltpu.make_async_copy(h_hbm.at[siw.at[0]], buf, sem).wait()

            def scat(j, buf):
                pltpu.sync_copy(buf, agg_sh.at[diw.at[j]], add=True)

            gath(0, buf0, gsem0)

            def step(i, carry):
                j = 2 * i
                gath(j + 1, buf1, gsem1)
                gwait(buf0, gsem0)
                scat(j, buf0)
                gath(j + 2, buf0, gsem0)
                gwait(buf1, gsem1)
                scat(j + 1, buf1)
                return carry

            lax.fori_loop(0, (_W - 1) // 2, step, 0)
            gwait(buf0, gsem0)
            scat(_W - 1, buf0)
        plsc.subcore_barrier()
        pltpu.sync_copy(agg_sh.at[pl.ds(s * _ROWS_PT, _ROWS_PT)],
                        out_hbm.at[c, pl.ds(s * _ROWS_PT, _ROWS_PT)])

    return body(h, src_r, dst_r, zeros_blk)


_BLK = 2048


def _mm_relu(x, W, b, out_dtype=jnp.float32):
    def body(x_ref, w_ref, b_ref, o_ref):
        acc = lax.dot_general(x_ref[...], w_ref[...],
                              (((1,), (1,)), ((), ())),
                              preferred_element_type=jnp.float32)
        o_ref[...] = jnp.maximum(acc + b_ref[...], 0.0).astype(out_dtype)

    return pl.pallas_call(
        body,
        grid=(_NP // _BLK,),
        in_specs=[
            pl.BlockSpec((_BLK, _D), lambda i: (i, 0)),
            pl.BlockSpec((_H, _D), lambda i: (0, 0)),
            pl.BlockSpec((1, _H), lambda i: (0, 0)),
        ],
        out_specs=pl.BlockSpec((_BLK, _H), lambda i: (i, 0)),
        out_shape=jax.ShapeDtypeStruct((_NP, _H), out_dtype),
    )(x, W, b.reshape(1, _H))


def _mm_relu_sum(p, W, b, out_dtype=jnp.float32):
    def body(p_ref, w_ref, b_ref, o_ref):
        a = (p_ref[0].astype(jnp.float32) + p_ref[1].astype(jnp.float32))
        acc = lax.dot_general(a, w_ref[...],
                              (((1,), (1,)), ((), ())),
                              preferred_element_type=jnp.float32)
        o_ref[...] = jnp.maximum(acc + b_ref[...], 0.0).astype(out_dtype)

    return pl.pallas_call(
        body,
        grid=(_NP // _BLK,),
        in_specs=[
            pl.BlockSpec((2, _BLK, _H), lambda i: (0, i, 0)),
            pl.BlockSpec((_H, _H), lambda i: (0, 0)),
            pl.BlockSpec((1, _H), lambda i: (0, 0)),
        ],
        out_specs=pl.BlockSpec((_BLK, _H), lambda i: (i, 0)),
        out_shape=jax.ShapeDtypeStruct((_NP, _H), out_dtype),
    )(p, W, b.reshape(1, _H))


def _mm_relu_sum_pool_heads(p, W, b, batch_row, Wm, bm, Wl, bl):
    nblk = _NP // _BLK

    def body(p_ref, w_ref, b_ref, br_ref, wm_ref, bm_ref, wl_ref, bl_ref,
             om_ref, ol_ref, acc_ref):
        i = pl.program_id(0)
        a = p_ref[0] + p_ref[1]
        h = jnp.maximum(
            lax.dot_general(a, w_ref[...], (((1,), (1,)), ((), ())),
                            preferred_element_type=jnp.float32)
            + b_ref[...], 0.0)
        gids = lax.broadcasted_iota(jnp.int32, (_NG, _BLK), 0)
        mask = jnp.where(br_ref[...] == gids, 1.0, 0.0)
        part = lax.dot_general(mask, h, (((1,), (0,)), ((), ())),
                               preferred_element_type=jnp.float32)

        @pl.when(i == 0)
        def _():
            acc_ref[...] = part

        @pl.when(i > 0)
        def _():
            acc_ref[...] += part

        @pl.when(i == nblk - 1)
        def _():
            pooled = acc_ref[...]
            om_ref[...] = lax.dot_general(
                pooled, wm_ref[...], (((1,), (1,)), ((), ())),
                preferred_element_type=jnp.float32) + bm_ref[...]
            ol_ref[...] = lax.dot_general(
                pooled, wl_ref[...], (((1,), (1,)), ((), ())),
                preferred_element_type=jnp.float32) + bl_ref[...]

    return pl.pallas_call(
        body,
        grid=(nblk,),
        in_specs=[
            pl.BlockSpec((2, _BLK, _H), lambda i: (0, i, 0)),
            pl.BlockSpec((_H, _H), lambda i: (0, 0)),
            pl.BlockSpec((1, _H), lambda i: (0, 0)),
            pl.BlockSpec((1, _BLK), lambda i: (0, i)),
            pl.BlockSpec((_L, _H), lambda i: (0, 0)),
            pl.BlockSpec((1, _L), lambda i: (0, 0)),
            pl.BlockSpec((_L, _H), lambda i: (0, 0)),
            pl.BlockSpec((1, _L), lambda i: (0, 0)),
        ],
        out_specs=[pl.BlockSpec((_NG, _L), lambda i: (0, 0)),
                   pl.BlockSpec((_NG, _L), lambda i: (0, 0))],
        out_shape=[jax.ShapeDtypeStruct((_NG, _L), jnp.float32),
                   jax.ShapeDtypeStruct((_NG, _L), jnp.float32)],
        scratch_shapes=[pltpu.VMEM((_NG, _H), jnp.float32)],
    )(p, W, b.reshape(1, _H), batch_row, Wm, bm.reshape(1, _L),
      Wl, bl.reshape(1, _L))


def kernel(x, edge_index, batch, W_in, b_in, W1, b1, W2, b2, W3, b3,
           W_mean, b_mean, W_logvar, b_logvar):
    src_r = edge_index[0].reshape(_NTILE * _NW, _W, _K)
    dst_r = edge_index[1].reshape(_NTILE * _NW, _W, _K)
    zeros_blk = jnp.zeros((_ROWS_PT, _H), dtype=jnp.float32)
    x_pad = jnp.pad(x, ((0, _NP - _N), (0, 0)))
    batch_row = jnp.pad(batch.astype(jnp.int32), (0, _NP - _N),
                        constant_values=_NG).reshape(1, _NP)

    h = _mm_relu(x_pad, W_in, b_in)
    for W, b in ((W1, b1), (W2, b2)):
        p = _mp_round(h, src_r, dst_r, zeros_blk)
        h = _mm_relu_sum(p, W, b)
    p = _mp_round(h, src_r, dst_r, zeros_blk)
    z_mean, z_logvar = _mm_relu_sum_pool_heads(
        p, W3, b3, batch_row, W_mean, b_mean, W_logvar, b_logvar)
    return (z_mean, z_logvar)
